# baseline (speedup 1.0000x reference)
import functools

import jax
import jax.numpy as jnp
from jax.experimental import pallas as pl
from jax.experimental.pallas import tpu as pltpu


def _linear_kernel(xl_ref, xr_ref, wt_ref, wb_ref, b_ref, o_ref):
    acc = (
        jnp.dot(xl_ref[...], wt_ref[...], preferred_element_type=jnp.float32)
        + jnp.dot(xr_ref[...], wb_ref[...], preferred_element_type=jnp.float32)
        + b_ref[...]
    )
    o_ref[...] = acc[:, : o_ref.shape[1]]


@functools.partial(jax.jit, static_argnames=("output_size", "tm"))
def _forward(bow_vec, w_p, b_p, *, output_size, tm):
    B, F = bow_vec.shape
    F_pad, O_pad = w_p.shape
    tk = F_pad // 2

    w_hbm = pltpu.with_memory_space_constraint(w_p, pltpu.MemorySpace.HBM)
    b_hbm = pltpu.with_memory_space_constraint(b_p, pltpu.MemorySpace.HBM)
    return pl.pallas_call(
        _linear_kernel,
        out_shape=jax.ShapeDtypeStruct((B, output_size), jnp.float32),
        grid=(B // tm,),
        in_specs=[
            pl.BlockSpec((tm, tk), lambda i: (i, 0)),
            pl.BlockSpec((tm, tk), lambda i: (i, 1)),
            pl.BlockSpec((tk, O_pad), lambda i: (0, 0)),
            pl.BlockSpec((tk, O_pad), lambda i: (1, 0)),
            pl.BlockSpec((1, O_pad), lambda i: (0, 0)),
        ],
        out_specs=pl.BlockSpec((tm, output_size), lambda i: (i, 0)),
        compiler_params=pltpu.CompilerParams(
            dimension_semantics=("arbitrary",),
            vmem_limit_bytes=48 * 1024 * 1024,
        ),
    )(bow_vec, bow_vec, w_hbm, w_hbm, b_hbm)


def kernel(bow_vec, w_p, b_p):
    return _forward(bow_vec, w_p, b_p, output_size=100, tm=512)


# speedup vs baseline: 1.0108x; 1.0108x over previous
import functools

import jax
import jax.numpy as jnp
from jax.experimental import pallas as pl
from jax.experimental.pallas import tpu as pltpu


def _linear_kernel(x_ref, w_ref, b_ref, o_ref):
    acc = jnp.dot(x_ref[...], w_ref[...],
                  preferred_element_type=jnp.float32) + b_ref[...]
    o_ref[...] = acc[:, : o_ref.shape[1]]


@functools.partial(jax.jit, static_argnames=("output_size", "tm"))
def _forward(bow_vec, w_p, b_p, *, output_size, tm):
    B, F = bow_vec.shape
    F_pad, O_pad = w_p.shape

    w_hbm = pltpu.with_memory_space_constraint(w_p, pltpu.MemorySpace.HBM)
    b_hbm = pltpu.with_memory_space_constraint(b_p, pltpu.MemorySpace.HBM)
    return pl.pallas_call(
        _linear_kernel,
        out_shape=jax.ShapeDtypeStruct((B, output_size), jnp.float32),
        grid=(B // tm,),
        in_specs=[
            pl.BlockSpec((tm, F_pad), lambda i: (i, 0)),
            pl.BlockSpec((F_pad, O_pad), lambda i: (0, 0)),
            pl.BlockSpec((1, O_pad), lambda i: (0, 0)),
        ],
        out_specs=pl.BlockSpec((tm, output_size), lambda i: (i, 0)),
        compiler_params=pltpu.CompilerParams(
            dimension_semantics=("arbitrary",),
            vmem_limit_bytes=48 * 1024 * 1024,
        ),
    )(bow_vec, w_hbm, b_hbm)


def kernel(bow_vec, w_p, b_p):
    return _forward(bow_vec, w_p, b_p, output_size=100, tm=512)
